# baseline (speedup 1.0000x reference)
import functools

import jax
import jax.numpy as jnp
from jax import lax
from jax.experimental import pallas as pl
from jax.experimental.pallas import tpu as pltpu
from jax.experimental.pallas import tpu_sc as plsc

N_NODES = 10000
N_EDGES = 320000
D_FEAT = 128
NDEG = 6
NUM_GRAPHS = 64

NC, NS = 2, 16
NW = NC * NS
EPW = N_EDGES // NW
CHUNKA = 64
NCHUNKA = 157
EPW_PAD = CHUNKA * NCHUNKA
NPAD = 10112
RPT = NPAD // NS

BLK = 1000


def _sc_agg_body(table, srcdst4, z128, out, sdidx, rows, acc, sem):
    cid = lax.axis_index("c")
    sid = lax.axis_index("s")
    wid = cid * NS + sid
    r0 = sid * RPT

    pltpu.sync_copy(z128.at[pl.ds(r0, RPT)], acc.at[pl.ds(r0, RPT)])
    pltpu.sync_copy(srcdst4.at[wid], sdidx)
    plsc.subcore_barrier()

    @pl.loop(0, NCHUNKA)
    def _edge_chunk(j):
        pltpu.async_copy(table.at[sdidx.at[0, j]], rows, sem).wait()
        pltpu.sync_copy(rows, acc.at[sdidx.at[1, j]], add=True)

    plsc.subcore_barrier()
    pltpu.sync_copy(acc.at[pl.ds(r0, RPT)], out.at[cid, pl.ds(r0, RPT)])


def _sc_agg1_body(table, srcdst4, z128, ones128, out, degout,
                  sdidx, rows, acc, sem):
    cid = lax.axis_index("c")
    sid = lax.axis_index("s")
    wid = cid * NS + sid
    r0 = sid * RPT

    pltpu.sync_copy(z128.at[pl.ds(r0, RPT)], acc.at[pl.ds(r0, RPT)])
    pltpu.sync_copy(srcdst4.at[wid], sdidx)
    pltpu.sync_copy(ones128, rows)
    plsc.subcore_barrier()

    @pl.loop(0, NCHUNKA)
    def _deg_chunk(j):
        pltpu.sync_copy(rows, acc.at[sdidx.at[1, j]], add=True)

    plsc.subcore_barrier()
    pltpu.sync_copy(acc.at[pl.ds(r0, RPT)], degout.at[cid, pl.ds(r0, RPT)])
    pltpu.sync_copy(z128.at[pl.ds(r0, RPT)], acc.at[pl.ds(r0, RPT)])
    plsc.subcore_barrier()

    @pl.loop(0, NCHUNKA)
    def _edge_chunk(j):
        pltpu.async_copy(table.at[sdidx.at[0, j]], rows, sem).wait()
        pltpu.sync_copy(rows, acc.at[sdidx.at[1, j]], add=True)

    plsc.subcore_barrier()
    pltpu.sync_copy(acc.at[pl.ds(r0, RPT)], out.at[cid, pl.ds(r0, RPT)])


def _sc_mesh():
    return plsc.VectorSubcoreMesh(core_axis_name="c", subcore_axis_name="s",
                                  num_cores=NC, num_subcores=NS)


def _sc_aggregate(table, srcdst4, z128):
    f32 = jnp.float32
    fn = pl.kernel(
        _sc_agg_body,
        out_type=jax.ShapeDtypeStruct((NC, NPAD, D_FEAT), f32),
        mesh=_sc_mesh(),
        scratch_types=[
            pltpu.VMEM((2, NCHUNKA, CHUNKA), jnp.int32),
            pltpu.VMEM((CHUNKA, D_FEAT), f32),
            pltpu.VMEM_SHARED((NPAD, D_FEAT), f32),
            pltpu.SemaphoreType.DMA,
        ])
    return fn(table, srcdst4, z128)


def _sc_agg1(table, srcdst4, z128, ones128):
    f32 = jnp.float32
    fn = pl.kernel(
        _sc_agg1_body,
        out_type=(jax.ShapeDtypeStruct((NC, NPAD, D_FEAT), f32),
                  jax.ShapeDtypeStruct((NC, NPAD, D_FEAT), f32)),
        mesh=_sc_mesh(),
        scratch_types=[
            pltpu.VMEM((2, NCHUNKA, CHUNKA), jnp.int32),
            pltpu.VMEM((CHUNKA, D_FEAT), f32),
            pltpu.VMEM_SHARED((NPAD, D_FEAT), f32),
            pltpu.SemaphoreType.DMA,
        ])
    return fn(table, srcdst4, z128, ones128)


def _mf_body(agg_ref, x_ref, dcnt_ref, wl_ref, wr_ref, b_ref, out_ref):
    h = agg_ref[0] + agg_ref[1]
    r = jnp.dot(h, wl_ref[...], preferred_element_type=jnp.float32)
    r = r + jnp.dot(x_ref[...], wr_ref[...], preferred_element_type=jnp.float32)
    r = r + b_ref[...]
    deg = jnp.minimum(dcnt_ref[0, :, 0:1] + dcnt_ref[1, :, 0:1], float(NDEG - 1))
    out = r[:, 0:D_FEAT]
    for d in range(1, NDEG):
        out = jnp.where(deg == float(d), r[:, d * D_FEAT:(d + 1) * D_FEAT], out)
    out_ref[...] = out


def _mf2_body(agg_ref, x_ref, dcnt_ref, wl_ref, wr_ref, b_ref, batch_ref,
              out_ref, pool_ref):
    h = agg_ref[0] + agg_ref[1]
    r = jnp.dot(h, wl_ref[...], preferred_element_type=jnp.float32)
    r = r + jnp.dot(x_ref[...], wr_ref[...], preferred_element_type=jnp.float32)
    r = r + b_ref[...]
    deg = jnp.minimum(dcnt_ref[0, :, 0:1] + dcnt_ref[1, :, 0:1], float(NDEG - 1))
    out = r[:, 0:D_FEAT]
    for d in range(1, NDEG):
        out = jnp.where(deg == float(d), r[:, d * D_FEAT:(d + 1) * D_FEAT], out)
    out_ref[...] = out

    @pl.when(pl.program_id(0) == 0)
    def _():
        pool_ref[...] = jnp.zeros_like(pool_ref)

    onehot = (batch_ref[...] ==
              lax.broadcasted_iota(jnp.int32, (BLK, NUM_GRAPHS), 1)
              ).astype(jnp.float32)
    pool_ref[...] += lax.dot_general(
        onehot, out, (((0,), (0,)), ((), ())),
        preferred_element_type=jnp.float32,
        precision=lax.Precision.HIGHEST)


def _fin_body(h1_ref, h2_ref, batch_ref, pool_ref, wf_ref, bf_ref, out_ref):
    onehot = (batch_ref[...] ==
              lax.broadcasted_iota(jnp.int32, (BLK, NUM_GRAPHS), 1)
              ).astype(jnp.float32)
    rep = jnp.dot(onehot, pool_ref[...], preferred_element_type=jnp.float32,
                  precision=lax.Precision.HIGHEST)
    logits = jnp.dot(h1_ref[...], wf_ref[0:D_FEAT, :],
                     preferred_element_type=jnp.float32)
    logits += jnp.dot(h2_ref[...], wf_ref[D_FEAT:2 * D_FEAT, :],
                      preferred_element_type=jnp.float32)
    logits += jnp.dot(rep, wf_ref[2 * D_FEAT:3 * D_FEAT, :],
                      preferred_element_type=jnp.float32)
    logits += bf_ref[...]
    m = jnp.max(logits, axis=1, keepdims=True)
    e = jnp.exp(logits - m)
    out_ref[...] = e / jnp.sum(e, axis=1, keepdims=True)


def _mfconv_tc(agg, x, dcnt, wcat_l, wcat_r, bcat, batch2d=None):
    grid = (N_NODES // BLK,)
    f32 = jnp.float32
    common_in = [
        pl.BlockSpec((NC, BLK, D_FEAT), lambda i: (0, i, 0)),
        pl.BlockSpec((BLK, D_FEAT), lambda i: (i, 0)),
        pl.BlockSpec((NC, BLK, D_FEAT), lambda i: (0, i, 0)),
        pl.BlockSpec((D_FEAT, NDEG * D_FEAT), lambda i: (0, 0)),
        pl.BlockSpec((D_FEAT, NDEG * D_FEAT), lambda i: (0, 0)),
        pl.BlockSpec((1, NDEG * D_FEAT), lambda i: (0, 0)),
    ]
    if batch2d is None:
        return pl.pallas_call(
            _mf_body,
            grid=grid,
            in_specs=common_in,
            out_specs=pl.BlockSpec((BLK, D_FEAT), lambda i: (i, 0)),
            out_shape=jax.ShapeDtypeStruct((N_NODES, D_FEAT), f32),
        )(agg, x, dcnt, wcat_l, wcat_r, bcat)
    return pl.pallas_call(
        _mf2_body,
        grid=grid,
        in_specs=common_in + [pl.BlockSpec((BLK, 1), lambda i: (i, 0))],
        out_specs=[
            pl.BlockSpec((BLK, D_FEAT), lambda i: (i, 0)),
            pl.BlockSpec((NUM_GRAPHS, D_FEAT), lambda i: (0, 0)),
        ],
        out_shape=[
            jax.ShapeDtypeStruct((N_NODES, D_FEAT), f32),
            jax.ShapeDtypeStruct((NUM_GRAPHS, D_FEAT), f32),
        ],
    )(agg, x, dcnt, wcat_l, wcat_r, bcat, batch2d)


def _final_tc(h1, h2, batch2d, pool, wf, bf2d):
    grid = (N_NODES // BLK,)
    ncls = wf.shape[1]
    return pl.pallas_call(
        _fin_body,
        grid=grid,
        in_specs=[
            pl.BlockSpec((BLK, D_FEAT), lambda i: (i, 0)),
            pl.BlockSpec((BLK, D_FEAT), lambda i: (i, 0)),
            pl.BlockSpec((BLK, 1), lambda i: (i, 0)),
            pl.BlockSpec((NUM_GRAPHS, D_FEAT), lambda i: (0, 0)),
            pl.BlockSpec(wf.shape, lambda i: (0, 0)),
            pl.BlockSpec((1, ncls), lambda i: (0, 0)),
        ],
        out_specs=pl.BlockSpec((BLK, ncls), lambda i: (i, 0)),
        out_shape=jax.ShapeDtypeStruct((N_NODES, ncls), jnp.float32),
    )(h1, h2, batch2d, pool, wf, bf2d)


def _wcat(w):
    return jnp.transpose(w, (1, 0, 2)).reshape(w.shape[1], NDEG * w.shape[2])


def kernel(x, edge_index, batch, W1_l, b1_l, W1_r, W2_l, b2_l, W2_r, Wf, bf):
    f32 = jnp.float32
    npadw = EPW_PAD - EPW
    sink = N_NODES + (jnp.arange(npadw, dtype=jnp.int32) % (NPAD - N_NODES))
    src_p = jnp.pad(edge_index[0].reshape(NW, EPW), ((0, 0), (0, npadw)))
    dst_p = jnp.concatenate(
        [edge_index[1].reshape(NW, EPW),
         jnp.broadcast_to(sink, (NW, npadw))], axis=1)
    srcdst4 = jnp.stack([src_p.reshape(NW, NCHUNKA, CHUNKA),
                         dst_p.reshape(NW, NCHUNKA, CHUNKA)], axis=1)
    z128 = jnp.zeros((NPAD, D_FEAT), f32)
    ones128 = jnp.zeros((CHUNKA, D_FEAT), f32).at[:, 0].set(1.0)
    batch2d = batch.reshape(N_NODES, 1)

    agg1, dcnt = _sc_agg1(x, srcdst4, z128, ones128)
    h1 = _mfconv_tc(agg1, x, dcnt, _wcat(W1_l), _wcat(W1_r),
                    b1_l.reshape(1, -1))
    agg2 = _sc_aggregate(h1, srcdst4, z128)
    h2, pool = _mfconv_tc(agg2, h1, dcnt, _wcat(W2_l), _wcat(W2_r),
                          b2_l.reshape(1, -1), batch2d)
    return _final_tc(h1, h2, batch2d, pool, Wf, bf.reshape(1, -1))


# speedup vs baseline: 1.2498x; 1.2498x over previous
import functools

import jax
import jax.numpy as jnp
from jax import lax
from jax.experimental import pallas as pl
from jax.experimental.pallas import tpu as pltpu
from jax.experimental.pallas import tpu_sc as plsc

N_NODES = 10000
N_EDGES = 320000
D_FEAT = 128
NDEG = 6
NUM_GRAPHS = 64

NC, NS = 2, 16
NW = NC * NS
EPW = N_EDGES // NW
CHUNKA = 80
NCHUNKA = EPW // CHUNKA
NPAD = 10112
RPT = NPAD // NS

BLK = 1000


def _sc_agg_body(table, srcdst4, z128, out, sdidx, rows, acc, sem):
    cid = lax.axis_index("c")
    sid = lax.axis_index("s")
    wid = cid * NS + sid
    r0 = sid * RPT

    pltpu.sync_copy(z128.at[pl.ds(r0, RPT)], acc.at[pl.ds(r0, RPT)])
    pltpu.sync_copy(srcdst4.at[wid], sdidx)
    plsc.subcore_barrier()

    @pl.loop(0, NCHUNKA)
    def _edge_chunk(j):
        pltpu.async_copy(table.at[sdidx.at[0, j]], rows, sem).wait()
        pltpu.sync_copy(rows, acc.at[sdidx.at[1, j]], add=True)

    plsc.subcore_barrier()
    pltpu.sync_copy(acc.at[pl.ds(r0, RPT)], out.at[cid, pl.ds(r0, RPT)])


def _sc_agg1_body(table, srcdst4, z128, ones128, out, degout,
                  sdidx, rows, acc, sem):
    cid = lax.axis_index("c")
    sid = lax.axis_index("s")
    wid = cid * NS + sid
    r0 = sid * RPT

    pltpu.sync_copy(z128.at[pl.ds(r0, RPT)], acc.at[pl.ds(r0, RPT)])
    pltpu.sync_copy(srcdst4.at[wid], sdidx)
    pltpu.sync_copy(ones128, rows)
    plsc.subcore_barrier()

    @pl.loop(0, NCHUNKA)
    def _deg_chunk(j):
        pltpu.sync_copy(rows, acc.at[sdidx.at[1, j]], add=True)

    plsc.subcore_barrier()
    pltpu.sync_copy(acc.at[pl.ds(r0, RPT)], degout.at[cid, pl.ds(r0, RPT)])
    pltpu.sync_copy(z128.at[pl.ds(r0, RPT)], acc.at[pl.ds(r0, RPT)])
    plsc.subcore_barrier()

    @pl.loop(0, NCHUNKA)
    def _edge_chunk(j):
        pltpu.async_copy(table.at[sdidx.at[0, j]], rows, sem).wait()
        pltpu.sync_copy(rows, acc.at[sdidx.at[1, j]], add=True)

    plsc.subcore_barrier()
    pltpu.sync_copy(acc.at[pl.ds(r0, RPT)], out.at[cid, pl.ds(r0, RPT)])


def _sc_mesh():
    return plsc.VectorSubcoreMesh(core_axis_name="c", subcore_axis_name="s",
                                  num_cores=NC, num_subcores=NS)


def _sc_aggregate(table, srcdst4, z128):
    f32 = jnp.float32
    fn = pl.kernel(
        _sc_agg_body,
        out_type=jax.ShapeDtypeStruct((NC, NPAD, D_FEAT), f32),
        mesh=_sc_mesh(),
        scratch_types=[
            pltpu.VMEM((2, NCHUNKA, CHUNKA), jnp.int32),
            pltpu.VMEM((CHUNKA, D_FEAT), f32),
            pltpu.VMEM_SHARED((NPAD, D_FEAT), f32),
            pltpu.SemaphoreType.DMA,
        ])
    return fn(table, srcdst4, z128)


def _sc_agg1(table, srcdst4, z128, ones128):
    f32 = jnp.float32
    fn = pl.kernel(
        _sc_agg1_body,
        out_type=(jax.ShapeDtypeStruct((NC, NPAD, D_FEAT), f32),
                  jax.ShapeDtypeStruct((NC, NPAD, D_FEAT), f32)),
        mesh=_sc_mesh(),
        scratch_types=[
            pltpu.VMEM((2, NCHUNKA, CHUNKA), jnp.int32),
            pltpu.VMEM((CHUNKA, D_FEAT), f32),
            pltpu.VMEM_SHARED((NPAD, D_FEAT), f32),
            pltpu.SemaphoreType.DMA,
        ])
    return fn(table, srcdst4, z128, ones128)


def _mf_body(agg_ref, x_ref, dcnt_ref, wl_ref, wr_ref, b_ref, out_ref):
    h = agg_ref[0] + agg_ref[1]
    r = jnp.dot(h, wl_ref[...], preferred_element_type=jnp.float32)
    r = r + jnp.dot(x_ref[...], wr_ref[...], preferred_element_type=jnp.float32)
    r = r + b_ref[...]
    deg = jnp.minimum(dcnt_ref[0, :, 0:1] + dcnt_ref[1, :, 0:1], float(NDEG - 1))
    out = r[:, 0:D_FEAT]
    for d in range(1, NDEG):
        out = jnp.where(deg == float(d), r[:, d * D_FEAT:(d + 1) * D_FEAT], out)
    out_ref[...] = out


def _mf2_body(agg_ref, x_ref, dcnt_ref, wl_ref, wr_ref, b_ref, batch_ref,
              out_ref, pool_ref):
    h = agg_ref[0] + agg_ref[1]
    r = jnp.dot(h, wl_ref[...], preferred_element_type=jnp.float32)
    r = r + jnp.dot(x_ref[...], wr_ref[...], preferred_element_type=jnp.float32)
    r = r + b_ref[...]
    deg = jnp.minimum(dcnt_ref[0, :, 0:1] + dcnt_ref[1, :, 0:1], float(NDEG - 1))
    out = r[:, 0:D_FEAT]
    for d in range(1, NDEG):
        out = jnp.where(deg == float(d), r[:, d * D_FEAT:(d + 1) * D_FEAT], out)
    out_ref[...] = out

    @pl.when(pl.program_id(0) == 0)
    def _():
        pool_ref[...] = jnp.zeros_like(pool_ref)

    onehot = (batch_ref[...] ==
              lax.broadcasted_iota(jnp.int32, (BLK, NUM_GRAPHS), 1)
              ).astype(jnp.float32)
    pool_ref[...] += lax.dot_general(
        onehot, out, (((0,), (0,)), ((), ())),
        preferred_element_type=jnp.float32,
        precision=lax.Precision.HIGHEST)


def _fin_body(h1_ref, h2_ref, batch_ref, pool_ref, wf_ref, bf_ref, out_ref):
    onehot = (batch_ref[...] ==
              lax.broadcasted_iota(jnp.int32, (BLK, NUM_GRAPHS), 1)
              ).astype(jnp.float32)
    rep = jnp.dot(onehot, pool_ref[...], preferred_element_type=jnp.float32,
                  precision=lax.Precision.HIGHEST)
    logits = jnp.dot(h1_ref[...], wf_ref[0:D_FEAT, :],
                     preferred_element_type=jnp.float32)
    logits += jnp.dot(h2_ref[...], wf_ref[D_FEAT:2 * D_FEAT, :],
                      preferred_element_type=jnp.float32)
    logits += jnp.dot(rep, wf_ref[2 * D_FEAT:3 * D_FEAT, :],
                      preferred_element_type=jnp.float32)
    logits += bf_ref[...]
    m = jnp.max(logits, axis=1, keepdims=True)
    e = jnp.exp(logits - m)
    out_ref[...] = e / jnp.sum(e, axis=1, keepdims=True)


def _mfconv_tc(agg, x, dcnt, wcat_l, wcat_r, bcat, batch2d=None):
    grid = (N_NODES // BLK,)
    f32 = jnp.float32
    common_in = [
        pl.BlockSpec((NC, BLK, D_FEAT), lambda i: (0, i, 0)),
        pl.BlockSpec((BLK, D_FEAT), lambda i: (i, 0)),
        pl.BlockSpec((NC, BLK, D_FEAT), lambda i: (0, i, 0)),
        pl.BlockSpec((D_FEAT, NDEG * D_FEAT), lambda i: (0, 0)),
        pl.BlockSpec((D_FEAT, NDEG * D_FEAT), lambda i: (0, 0)),
        pl.BlockSpec((1, NDEG * D_FEAT), lambda i: (0, 0)),
    ]
    if batch2d is None:
        return pl.pallas_call(
            _mf_body,
            grid=grid,
            in_specs=common_in,
            out_specs=pl.BlockSpec((BLK, D_FEAT), lambda i: (i, 0)),
            out_shape=jax.ShapeDtypeStruct((N_NODES, D_FEAT), f32),
        )(agg, x, dcnt, wcat_l, wcat_r, bcat)
    return pl.pallas_call(
        _mf2_body,
        grid=grid,
        in_specs=common_in + [pl.BlockSpec((BLK, 1), lambda i: (i, 0))],
        out_specs=[
            pl.BlockSpec((BLK, D_FEAT), lambda i: (i, 0)),
            pl.BlockSpec((NUM_GRAPHS, D_FEAT), lambda i: (0, 0)),
        ],
        out_shape=[
            jax.ShapeDtypeStruct((N_NODES, D_FEAT), f32),
            jax.ShapeDtypeStruct((NUM_GRAPHS, D_FEAT), f32),
        ],
    )(agg, x, dcnt, wcat_l, wcat_r, bcat, batch2d)


def _final_tc(h1, h2, batch2d, pool, wf, bf2d):
    grid = (N_NODES // BLK,)
    ncls = wf.shape[1]
    return pl.pallas_call(
        _fin_body,
        grid=grid,
        in_specs=[
            pl.BlockSpec((BLK, D_FEAT), lambda i: (i, 0)),
            pl.BlockSpec((BLK, D_FEAT), lambda i: (i, 0)),
            pl.BlockSpec((BLK, 1), lambda i: (i, 0)),
            pl.BlockSpec((NUM_GRAPHS, D_FEAT), lambda i: (0, 0)),
            pl.BlockSpec(wf.shape, lambda i: (0, 0)),
            pl.BlockSpec((1, ncls), lambda i: (0, 0)),
        ],
        out_specs=pl.BlockSpec((BLK, ncls), lambda i: (i, 0)),
        out_shape=jax.ShapeDtypeStruct((N_NODES, ncls), jnp.float32),
    )(h1, h2, batch2d, pool, wf, bf2d)


def _wcat(w):
    return jnp.transpose(w, (1, 0, 2)).reshape(w.shape[1], NDEG * w.shape[2])


def kernel(x, edge_index, batch, W1_l, b1_l, W1_r, W2_l, b2_l, W2_r, Wf, bf):
    f32 = jnp.float32
    srcdst4 = jnp.stack([edge_index[0].reshape(NW, NCHUNKA, CHUNKA),
                         edge_index[1].reshape(NW, NCHUNKA, CHUNKA)], axis=1)
    z128 = jnp.zeros((NPAD, D_FEAT), f32)
    ones128 = jnp.zeros((CHUNKA, D_FEAT), f32).at[:, 0].set(1.0)
    batch2d = batch.reshape(N_NODES, 1)

    agg1, dcnt = _sc_agg1(x, srcdst4, z128, ones128)
    h1 = _mfconv_tc(agg1, x, dcnt, _wcat(W1_l), _wcat(W1_r),
                    b1_l.reshape(1, -1))
    agg2 = _sc_aggregate(h1, srcdst4, z128)
    h2, pool = _mfconv_tc(agg2, h1, dcnt, _wcat(W2_l), _wcat(W2_r),
                          b2_l.reshape(1, -1), batch2d)
    return _final_tc(h1, h2, batch2d, pool, Wf, bf.reshape(1, -1))
